# baseline (device time: 169816 ns/iter reference)
import jax
import jax.numpy as jnp
from jax import lax
from jax.experimental import pallas as pl
from jax.experimental.pallas import tpu as pltpu

N_DEV = 4
S = 4
MESH = pl.DeviceIdType.MESH


def kernel(x, w_mat):
    m, k = x.shape
    _, n = w_mat.shape
    mc = m // N_DEV
    nh = n // 2
    hc = mc // S

    def body(x_ref, w_ref, out_ref, res, slot_p, slot_m, xbuf, xb16, wb,
             rs_send_p, rs_recv_p, rs_send_m, rs_recv_m,
             ag_send_p, ag_recv_p, ag_send_m, ag_recv_m,
             x_sems, st_sems):
        my = lax.axis_index("i")
        left = lax.rem(my + (N_DEV - 1), N_DEV)
        right = lax.rem(my + 1, N_DEV)

        barrier = pltpu.get_barrier_semaphore()
        for nbr in (left, right):
            pl.semaphore_signal(
                barrier, inc=1, device_id=(nbr,), device_id_type=MESH,
            )

        def base(c):
            return lax.rem(c + 4 * N_DEV, N_DEV) * mc

        def rows(c):
            return pl.ds(base(c), mc)

        def rsub(c, j):
            return pl.ds(base(c) + j * hc, hc)

        f32 = jnp.float32
        bf16 = jnp.bfloat16
        L = slice(0, nh)
        R = slice(nh, n)

        def mk(src, dst, ssem, rsem, dev):
            return pltpu.make_async_remote_copy(
                src_ref=src, dst_ref=dst, send_sem=ssem, recv_sem=rsem,
                device_id=(dev,), device_id_type=MESH)

        rs_p, rs_m, ag_p, ag_m = [], [], [], []
        for h in range(N_DEV - 1):
            rs_p.append([mk(res.at[rsub(my - h, j), L],
                            slot_p.at[h, slice(j * hc, (j + 1) * hc)],
                            rs_send_p.at[h, j], rs_recv_p.at[h, j], right)
                         for j in range(S)])
            rs_m.append([mk(res.at[rsub(my + h, j), R],
                            slot_m.at[h, slice(j * hc, (j + 1) * hc)],
                            rs_send_m.at[h, j], rs_recv_m.at[h, j], left)
                         for j in range(S)])
            ag_p.append([mk(res.at[rsub(my + 1 - h, j), L],
                            res.at[rsub(my + 1 - h, j), L],
                            ag_send_p.at[h, j], ag_recv_p.at[h, j], right)
                         for j in range(S)])
            ag_m.append([mk(res.at[rsub(my - 1 + h, j), R],
                            res.at[rsub(my - 1 + h, j), R],
                            ag_send_m.at[h, j], ag_recv_m.at[h, j], left)
                         for j in range(S)])

        def acc(ring_slot, h, j, c, csl):
            sl = rsub(c, j)
            ssl = slice(j * hc, (j + 1) * hc)
            res[sl, csl] = res[sl, csl] + ring_slot[h, ssl]

        def xload(s, c):
            d = pltpu.make_async_copy(
                x_ref.at[rows(c), :], xbuf.at[s], x_sems.at[s])
            d.start()
            return d

        st_pending = [None] * 4
        st_ctr = [0]

        def store(sl, csl):
            i = st_ctr[0] % 4
            st_ctr[0] += 1
            if st_pending[i] is not None:
                st_pending[i].wait()
            d = pltpu.make_async_copy(
                res.at[sl, csl], out_ref.at[sl, csl], st_sems.at[i])
            d.start()
            st_pending[i] = d

        ld = xload(0, my)
        wb[:, L] = w_ref[:, L].astype(bf16)
        ld.wait()
        ld1 = xload(1, my - 1)
        xb16[0] = xbuf[0].astype(bf16)
        res[rsub(my, 0), L] = jnp.dot(
            xb16[0, : hc], wb[:, L], preferred_element_type=f32).astype(bf16)
        pl.semaphore_wait(barrier, 2)
        rs_p[0][0].start()
        res[pl.ds(base(my) + hc, mc - hc), L] = jnp.dot(
            xb16[0, hc:], wb[:, L], preferred_element_type=f32).astype(bf16)
        for j in range(1, S):
            rs_p[0][j].start()
        wb[:, R] = w_ref[:, R].astype(bf16)
        res[rows(my), R] = jnp.dot(
            xb16[0], wb[:, R], preferred_element_type=f32).astype(bf16)
        for j in range(S):
            rs_m[0][j].start()

        def gemm_full(s, c):
            xb16[s] = xbuf[s].astype(bf16)
            res[rows(c), L] = jnp.dot(
                xb16[s], wb[:, L], preferred_element_type=f32).astype(bf16)
            res[rows(c), R] = jnp.dot(
                xb16[s], wb[:, R], preferred_element_type=f32).astype(bf16)

        ld1.wait()
        ld2 = xload(0, my + 1)
        gemm_full(1, my - 1)
        ld2.wait()
        ld3 = xload(1, my + 2)
        gemm_full(0, my + 1)
        ld3.wait()
        gemm_full(1, my + 2)

        for h in range(N_DEV - 1):
            for j in range(S):
                rs_p[h][j].wait()
                acc(slot_p, h, j, my - h - 1, L)
                if h < N_DEV - 2:
                    rs_p[h + 1][j].start()
                else:
                    ag_p[0][j].start()
                rs_m[h][j].wait()
                acc(slot_m, h, j, my + h + 1, R)
                if h < N_DEV - 2:
                    rs_m[h + 1][j].start()
                else:
                    ag_m[0][j].start()
                    if j == S - 1:
                        store(rows(my + 1), L)
                        store(rows(my - 1), R)

        for h in range(N_DEV - 1):
            for j in range(S):
                ag_p[h][j].wait_recv()
                if h < N_DEV - 2:
                    ag_p[h + 1][j].start()
                store(rsub(my - h, j), L)
                ag_m[h][j].wait_recv()
                if h < N_DEV - 2:
                    ag_m[h + 1][j].start()
                store(rsub(my + h, j), R)

        for d in st_pending:
            if d is not None:
                d.wait()
        for h in range(N_DEV - 1):
            for j in range(S):
                ag_p[h][j].wait_send()
                ag_m[h][j].wait_send()

    out_bf16 = pl.pallas_call(
        body,
        out_shape=jax.ShapeDtypeStruct((m, n), jnp.bfloat16),
        in_specs=[
            pl.BlockSpec(memory_space=pl.ANY),
            pl.BlockSpec(memory_space=pltpu.VMEM),
        ],
        out_specs=pl.BlockSpec(memory_space=pl.ANY),
        scratch_shapes=[
            pltpu.VMEM((m, n), jnp.bfloat16),
            pltpu.VMEM((N_DEV - 1, mc, nh), jnp.bfloat16),
            pltpu.VMEM((N_DEV - 1, mc, nh), jnp.bfloat16),
            pltpu.VMEM((2, mc, k), jnp.float32),
            pltpu.VMEM((2, mc, k), jnp.bfloat16),
            pltpu.VMEM((k, n), jnp.bfloat16),
            pltpu.SemaphoreType.DMA((N_DEV - 1, S)),
            pltpu.SemaphoreType.DMA((N_DEV - 1, S)),
            pltpu.SemaphoreType.DMA((N_DEV - 1, S)),
            pltpu.SemaphoreType.DMA((N_DEV - 1, S)),
            pltpu.SemaphoreType.DMA((N_DEV - 1, S)),
            pltpu.SemaphoreType.DMA((N_DEV - 1, S)),
            pltpu.SemaphoreType.DMA((N_DEV - 1, S)),
            pltpu.SemaphoreType.DMA((N_DEV - 1, S)),
            pltpu.SemaphoreType.DMA((2,)),
            pltpu.SemaphoreType.DMA((4,)),
        ],
        compiler_params=pltpu.CompilerParams(
            collective_id=0,
            vmem_limit_bytes=64 * 1024 * 1024,
        ),
    )(x, w_mat)
    return out_bf16.astype(jnp.float32)


# device time: 169531 ns/iter; 1.0017x vs baseline; 1.0017x over previous
import jax
import jax.numpy as jnp
from jax import lax
from jax.experimental import pallas as pl
from jax.experimental.pallas import tpu as pltpu

N_DEV = 4
S = 2
MESH = pl.DeviceIdType.MESH


def kernel(x, w_mat):
    m, k = x.shape
    _, n = w_mat.shape
    mc = m // N_DEV
    nh = n // 2
    hc = mc // S

    def body(x_ref, w_ref, out_ref, res, slot_p, slot_m, xbuf, xb16, wb,
             rs_send_p, rs_recv_p, rs_send_m, rs_recv_m,
             ag_send_p, ag_recv_p, ag_send_m, ag_recv_m,
             x_sems, st_sems):
        my = lax.axis_index("i")
        left = lax.rem(my + (N_DEV - 1), N_DEV)
        right = lax.rem(my + 1, N_DEV)

        barrier = pltpu.get_barrier_semaphore()
        for nbr in (left, right):
            pl.semaphore_signal(
                barrier, inc=1, device_id=(nbr,), device_id_type=MESH,
            )

        def base(c):
            return lax.rem(c + 4 * N_DEV, N_DEV) * mc

        def rows(c):
            return pl.ds(base(c), mc)

        def rsub(c, j):
            return pl.ds(base(c) + j * hc, hc)

        f32 = jnp.float32
        bf16 = jnp.bfloat16
        L = slice(0, nh)
        R = slice(nh, n)

        def mk(src, dst, ssem, rsem, dev):
            return pltpu.make_async_remote_copy(
                src_ref=src, dst_ref=dst, send_sem=ssem, recv_sem=rsem,
                device_id=(dev,), device_id_type=MESH)

        rs_p, rs_m, ag_p, ag_m = [], [], [], []
        for h in range(N_DEV - 1):
            rs_p.append([mk(res.at[rsub(my - h, j), L],
                            slot_p.at[h, slice(j * hc, (j + 1) * hc)],
                            rs_send_p.at[h, j], rs_recv_p.at[h, j], right)
                         for j in range(S)])
            rs_m.append([mk(res.at[rsub(my + h, j), R],
                            slot_m.at[h, slice(j * hc, (j + 1) * hc)],
                            rs_send_m.at[h, j], rs_recv_m.at[h, j], left)
                         for j in range(S)])
            ag_p.append([mk(res.at[rsub(my + 1 - h, j), L],
                            res.at[rsub(my + 1 - h, j), L],
                            ag_send_p.at[h, j], ag_recv_p.at[h, j], right)
                         for j in range(S)])
            ag_m.append([mk(res.at[rsub(my - 1 + h, j), R],
                            res.at[rsub(my - 1 + h, j), R],
                            ag_send_m.at[h, j], ag_recv_m.at[h, j], left)
                         for j in range(S)])

        def acc(ring_slot, h, j, c, csl):
            sl = rsub(c, j)
            ssl = slice(j * hc, (j + 1) * hc)
            res[sl, csl] = res[sl, csl] + ring_slot[h, ssl]

        def xload(s, c):
            d = pltpu.make_async_copy(
                x_ref.at[rows(c), :], xbuf.at[s], x_sems.at[s])
            d.start()
            return d

        st_pending = [None] * 4
        st_ctr = [0]

        def store(sl, csl):
            i = st_ctr[0] % 4
            st_ctr[0] += 1
            if st_pending[i] is not None:
                st_pending[i].wait()
            d = pltpu.make_async_copy(
                res.at[sl, csl], out_ref.at[sl, csl], st_sems.at[i])
            d.start()
            st_pending[i] = d

        ld = xload(0, my)
        wb[:, L] = w_ref[:, L].astype(bf16)
        ld.wait()
        ld1 = xload(1, my - 1)
        xb16[0] = xbuf[0].astype(bf16)
        res[rsub(my, 0), L] = jnp.dot(
            xb16[0, : hc], wb[:, L], preferred_element_type=f32).astype(bf16)
        pl.semaphore_wait(barrier, 2)
        rs_p[0][0].start()
        res[pl.ds(base(my) + hc, mc - hc), L] = jnp.dot(
            xb16[0, hc:], wb[:, L], preferred_element_type=f32).astype(bf16)
        for j in range(1, S):
            rs_p[0][j].start()
        wb[:, R] = w_ref[:, R].astype(bf16)
        res[rows(my), R] = jnp.dot(
            xb16[0], wb[:, R], preferred_element_type=f32).astype(bf16)
        for j in range(S):
            rs_m[0][j].start()

        def gemm_full(s, c):
            xb16[s] = xbuf[s].astype(bf16)
            res[rows(c), L] = jnp.dot(
                xb16[s], wb[:, L], preferred_element_type=f32).astype(bf16)
            res[rows(c), R] = jnp.dot(
                xb16[s], wb[:, R], preferred_element_type=f32).astype(bf16)

        ld1.wait()
        ld2 = xload(0, my + 1)
        gemm_full(1, my - 1)
        ld2.wait()
        ld3 = xload(1, my + 2)
        gemm_full(0, my + 1)
        ld3.wait()
        gemm_full(1, my + 2)

        for h in range(N_DEV - 1):
            for j in range(S):
                rs_p[h][j].wait()
                acc(slot_p, h, j, my - h - 1, L)
                if h < N_DEV - 2:
                    rs_p[h + 1][j].start()
                else:
                    ag_p[0][j].start()
                rs_m[h][j].wait()
                acc(slot_m, h, j, my + h + 1, R)
                if h < N_DEV - 2:
                    rs_m[h + 1][j].start()
                else:
                    ag_m[0][j].start()
                    if j == S - 1:
                        store(rows(my + 1), L)
                        store(rows(my - 1), R)

        for h in range(N_DEV - 1):
            for j in range(S):
                ag_p[h][j].wait_recv()
                if h < N_DEV - 2:
                    ag_p[h + 1][j].start()
                store(rsub(my - h, j), L)
                ag_m[h][j].wait_recv()
                if h < N_DEV - 2:
                    ag_m[h + 1][j].start()
                store(rsub(my + h, j), R)

        for d in st_pending:
            if d is not None:
                d.wait()
        for h in range(N_DEV - 1):
            for j in range(S):
                ag_p[h][j].wait_send()
                ag_m[h][j].wait_send()

    out_bf16 = pl.pallas_call(
        body,
        out_shape=jax.ShapeDtypeStruct((m, n), jnp.bfloat16),
        in_specs=[
            pl.BlockSpec(memory_space=pl.ANY),
            pl.BlockSpec(memory_space=pltpu.VMEM),
        ],
        out_specs=pl.BlockSpec(memory_space=pl.ANY),
        scratch_shapes=[
            pltpu.VMEM((m, n), jnp.bfloat16),
            pltpu.VMEM((N_DEV - 1, mc, nh), jnp.bfloat16),
            pltpu.VMEM((N_DEV - 1, mc, nh), jnp.bfloat16),
            pltpu.VMEM((2, mc, k), jnp.float32),
            pltpu.VMEM((2, mc, k), jnp.bfloat16),
            pltpu.VMEM((k, n), jnp.bfloat16),
            pltpu.SemaphoreType.DMA((N_DEV - 1, S)),
            pltpu.SemaphoreType.DMA((N_DEV - 1, S)),
            pltpu.SemaphoreType.DMA((N_DEV - 1, S)),
            pltpu.SemaphoreType.DMA((N_DEV - 1, S)),
            pltpu.SemaphoreType.DMA((N_DEV - 1, S)),
            pltpu.SemaphoreType.DMA((N_DEV - 1, S)),
            pltpu.SemaphoreType.DMA((N_DEV - 1, S)),
            pltpu.SemaphoreType.DMA((N_DEV - 1, S)),
            pltpu.SemaphoreType.DMA((2,)),
            pltpu.SemaphoreType.DMA((4,)),
        ],
        compiler_params=pltpu.CompilerParams(
            collective_id=0,
            vmem_limit_bytes=64 * 1024 * 1024,
        ),
    )(x, w_mat)
    return out_bf16.astype(jnp.float32)
